# baseline (speedup 1.0000x reference)
import jax
import jax.numpy as jnp
from jax.experimental import pallas as pl
from jax.experimental.pallas import tpu as pltpu


def _conf_kernel(pred_ref, tgt_ref, hist_ref, keys_ref, vals_ref,
                 acc_ref, col_ref, cls_ref):
    i = pl.program_id(0)
    nb = pl.num_programs(0)
    B, C = pred_ref.shape

    @pl.when(i == 0)
    def _init():
        acc_ref[...] = jnp.zeros_like(acc_ref)
        col_ref[...] = jax.lax.broadcasted_iota(
            jnp.int32, (B, C), 1).astype(jnp.float32)
        cls_ref[...] = jax.lax.broadcasted_iota(jnp.int32, (C, B), 0)

    col = col_ref[...]
    cls = cls_ref[...]
    one = jnp.float32(1.0)
    zero = jnp.float32(0.0)
    p = pred_ref[...]
    t = tgt_ref[pl.ds(i, 1), :]
    m = jnp.max(p, axis=1, keepdims=True)
    lab = jnp.min(jnp.where(p == m, col, float(C)), axis=1, keepdims=True)
    oh_l = jnp.where(col == lab, one, zero)
    oh_t = jnp.where(cls == t, one, zero)
    acc_ref[...] += jax.lax.dot_general(
        oh_t, oh_l, (((1,), (0,)), ((), ())),
        preferred_element_type=jnp.float32)

    @pl.when(i == nb - 1)
    def _finish():
        h = acc_ref[...]
        hist_ref[...] = h.astype(jnp.int32)
        r = jax.lax.broadcasted_iota(jnp.int32, (C, C), 0).astype(jnp.float32)
        c = jax.lax.broadcasted_iota(jnp.int32, (C, C), 1).astype(jnp.float32)
        eye = (r == c)
        dcol = jnp.sum(jnp.where(eye, h, 0.0), axis=1, keepdims=True)
        drow = jnp.sum(jnp.where(eye, h, 0.0), axis=0, keepdims=True)
        wins = (drow < dcol) | ((drow == dcol) & (c < r))
        rank = jnp.sum(jnp.where(wins, 1.0, 0.0), axis=1, keepdims=True)
        q = jnp.where(rank == c, 1.0, 0.0)
        vals_ref[...] = jnp.sum(q * dcol, axis=0, keepdims=True).astype(jnp.int32)
        keys_ref[...] = jnp.sum(q * r, axis=0, keepdims=True).astype(jnp.int32)


def _pick_block(n):
    if n * 4 > (8 << 20):
        return None
    best = None
    for b in range(8, 16385, 8):
        if n % b == 0:
            best = b
    return best


def kernel(pred, target):
    n, n_class = pred.shape
    b = _pick_block(n)
    if b is None:
        bp = 512
        npad = (n + bp - 1) // bp * bp
        pred = jnp.pad(pred, ((0, npad - n), (0, 0)))
        target = jnp.pad(target, (0, npad - n), constant_values=-1)
        n = npad
        b = _pick_block(n)
        if b is None:
            raise ValueError("unsupported input size")
    nb = n // b
    tgt2 = target.astype(jnp.int32).reshape(nb, b)
    hist, keys, vals = pl.pallas_call(
        _conf_kernel,
        grid=(nb,),
        in_specs=[
            pl.BlockSpec((b, n_class), lambda i: (i, 0)),
            pl.BlockSpec((nb, b), lambda i: (0, 0)),
        ],
        out_specs=[
            pl.BlockSpec((n_class, n_class), lambda i: (0, 0)),
            pl.BlockSpec((1, n_class), lambda i: (0, 0)),
            pl.BlockSpec((1, n_class), lambda i: (0, 0)),
        ],
        out_shape=[
            jax.ShapeDtypeStruct((n_class, n_class), jnp.int32),
            jax.ShapeDtypeStruct((1, n_class), jnp.int32),
            jax.ShapeDtypeStruct((1, n_class), jnp.int32),
        ],
        scratch_shapes=[
            pltpu.VMEM((n_class, n_class), jnp.float32),
            pltpu.VMEM((b, n_class), jnp.float32),
            pltpu.VMEM((n_class, b), jnp.int32),
        ],
        compiler_params=pltpu.CompilerParams(
            dimension_semantics=("arbitrary",),
            fuse_transposed_lhs_in_matmul=True),
    )(pred, tgt2)
    return hist, keys.reshape(n_class), vals.reshape(n_class)


# speedup vs baseline: 4.5721x; 4.5721x over previous
import jax
import jax.numpy as jnp
from jax.experimental import pallas as pl
from jax.experimental.pallas import tpu as pltpu


def _conf_kernel(predt_ref, tgt_ref, hist_ref, keys_ref, vals_ref,
                 acc_ref, cls_ref):
    i = pl.program_id(0)
    nb = pl.num_programs(0)
    C, B = predt_ref.shape

    @pl.when(i == 0)
    def _init():
        acc_ref[...] = jnp.zeros_like(acc_ref)
        cls_ref[...] = jax.lax.broadcasted_iota(
            jnp.int32, (C, B), 0).astype(jnp.float32)

    cls = cls_ref[...]
    one = jnp.float32(1.0)
    zero = jnp.float32(0.0)
    p = predt_ref[...]
    t = tgt_ref[pl.ds(i, 1), :].astype(jnp.float32)
    m = jnp.max(p, axis=0, keepdims=True)
    lab = jnp.min(jnp.where(p == m, cls, float(C)), axis=0, keepdims=True)
    oh_l = jnp.where(cls == lab, one, zero)
    oh_t = jnp.where(cls == t, one, zero)
    acc_ref[...] += jax.lax.dot_general(
        oh_t, oh_l, (((1,), (1,)), ((), ())),
        preferred_element_type=jnp.float32)

    @pl.when(i == nb - 1)
    def _finish():
        h = acc_ref[...]
        hist_ref[...] = h.astype(jnp.int32)
        r = jax.lax.broadcasted_iota(jnp.int32, (C, C), 0).astype(jnp.float32)
        c = jax.lax.broadcasted_iota(jnp.int32, (C, C), 1).astype(jnp.float32)
        eye = (r == c)
        dcol = jnp.sum(jnp.where(eye, h, 0.0), axis=1, keepdims=True)
        drow = jnp.sum(jnp.where(eye, h, 0.0), axis=0, keepdims=True)
        wins = (drow < dcol) | ((drow == dcol) & (c < r))
        rank = jnp.sum(jnp.where(wins, 1.0, 0.0), axis=1, keepdims=True)
        q = jnp.where(rank == c, 1.0, 0.0)
        vals_ref[...] = jnp.sum(q * dcol, axis=0, keepdims=True).astype(jnp.int32)
        keys_ref[...] = jnp.sum(q * r, axis=0, keepdims=True).astype(jnp.int32)


def kernel(pred, target):
    n, n_class = pred.shape
    b = 10240
    nb = (n + b - 1) // b
    npad = nb * b
    tgt2 = jnp.pad(target.astype(jnp.int32), (0, npad - n),
                   constant_values=-1).reshape(nb, b)
    predt = pred.T
    hist, keys, vals = pl.pallas_call(
        _conf_kernel,
        grid=(nb,),
        in_specs=[
            pl.BlockSpec((n_class, b), lambda i: (0, i)),
            pl.BlockSpec((nb, b), lambda i: (0, 0)),
        ],
        out_specs=[
            pl.BlockSpec((n_class, n_class), lambda i: (0, 0)),
            pl.BlockSpec((1, n_class), lambda i: (0, 0)),
            pl.BlockSpec((1, n_class), lambda i: (0, 0)),
        ],
        out_shape=[
            jax.ShapeDtypeStruct((n_class, n_class), jnp.int32),
            jax.ShapeDtypeStruct((1, n_class), jnp.int32),
            jax.ShapeDtypeStruct((1, n_class), jnp.int32),
        ],
        scratch_shapes=[
            pltpu.VMEM((n_class, n_class), jnp.float32),
            pltpu.VMEM((n_class, b), jnp.float32),
        ],
        compiler_params=pltpu.CompilerParams(
            dimension_semantics=("arbitrary",),
            fuse_transposed_lhs_in_matmul=True),
    )(predt, tgt2)
    return hist, keys.reshape(n_class), vals.reshape(n_class)


# speedup vs baseline: 5.3165x; 1.1628x over previous
import jax
import jax.numpy as jnp
from jax.experimental import pallas as pl
from jax.experimental.pallas import tpu as pltpu


def _conf_kernel(predt_ref, tgt_ref, hist_ref, keys_ref, vals_ref,
                 acc_ref, cls_ref):
    i = pl.program_id(0)
    nb = pl.num_programs(0)
    C, B = predt_ref.shape

    @pl.when(i == 0)
    def _init():
        acc_ref[...] = jnp.zeros_like(acc_ref)
        cls_ref[...] = jax.lax.broadcasted_iota(
            jnp.int32, (C, B), 0).astype(jnp.float32)

    cls = cls_ref[...]
    one = jnp.float32(1.0)
    zero = jnp.float32(0.0)
    p = predt_ref[...]
    t = tgt_ref[pl.ds(i, 1), :].astype(jnp.float32)
    m = jnp.max(p, axis=0, keepdims=True)
    lab = jnp.min(jnp.where(p == m, cls, float(C)), axis=0, keepdims=True)
    oh_l = jnp.where(cls == lab, one, zero)
    oh_t = jnp.where(cls == t, one, zero)
    acc_ref[...] += jax.lax.dot_general(
        oh_t, oh_l, (((1,), (1,)), ((), ())),
        preferred_element_type=jnp.float32)

    @pl.when(i == nb - 1)
    def _finish():
        h = acc_ref[...]
        hist_ref[...] = h.astype(jnp.int32)
        r = jax.lax.broadcasted_iota(jnp.int32, (C, C), 0).astype(jnp.float32)
        c = jax.lax.broadcasted_iota(jnp.int32, (C, C), 1).astype(jnp.float32)
        eye = (r == c)
        dcol = jnp.sum(jnp.where(eye, h, 0.0), axis=1, keepdims=True)
        drow = jnp.sum(jnp.where(eye, h, 0.0), axis=0, keepdims=True)
        wins = (drow < dcol) | ((drow == dcol) & (c < r))
        rank = jnp.sum(jnp.where(wins, 1.0, 0.0), axis=1, keepdims=True)
        q = jnp.where(rank == c, 1.0, 0.0)
        vals_ref[...] = jnp.sum(q * dcol, axis=0, keepdims=True).astype(jnp.int32)
        keys_ref[...] = jnp.sum(q * r, axis=0, keepdims=True).astype(jnp.int32)


def kernel(pred, target):
    n, n_class = pred.shape
    b = 16384
    nb = (n + b - 1) // b
    npad = nb * b
    tgt2 = jnp.pad(target.astype(jnp.int32), (0, npad - n),
                   constant_values=-1).reshape(nb, b)
    predt = pred.T
    hist, keys, vals = pl.pallas_call(
        _conf_kernel,
        grid=(nb,),
        in_specs=[
            pl.BlockSpec((n_class, b), lambda i: (0, i)),
            pl.BlockSpec((nb, b), lambda i: (0, 0)),
        ],
        out_specs=[
            pl.BlockSpec((n_class, n_class), lambda i: (0, 0)),
            pl.BlockSpec((1, n_class), lambda i: (0, 0)),
            pl.BlockSpec((1, n_class), lambda i: (0, 0)),
        ],
        out_shape=[
            jax.ShapeDtypeStruct((n_class, n_class), jnp.int32),
            jax.ShapeDtypeStruct((1, n_class), jnp.int32),
            jax.ShapeDtypeStruct((1, n_class), jnp.int32),
        ],
        scratch_shapes=[
            pltpu.VMEM((n_class, n_class), jnp.float32),
            pltpu.VMEM((n_class, b), jnp.float32),
        ],
        compiler_params=pltpu.CompilerParams(
            dimension_semantics=("arbitrary",),
            fuse_transposed_lhs_in_matmul=True),
    )(predt, tgt2)
    return hist, keys.reshape(n_class), vals.reshape(n_class)


# speedup vs baseline: 6.2738x; 1.1801x over previous
import jax
import jax.numpy as jnp
from jax.experimental import pallas as pl
from jax.experimental.pallas import tpu as pltpu


def _conf_kernel(predt_ref, tgt_ref, hist_ref, keys_ref, vals_ref,
                 acc_ref, cls_ref):
    i = pl.program_id(0)
    nb = pl.num_programs(0)
    C, B = predt_ref.shape

    @pl.when(i == 0)
    def _init():
        acc_ref[...] = jnp.zeros_like(acc_ref)
        cls_ref[...] = jax.lax.broadcasted_iota(
            jnp.int32, (C, B), 0).astype(jnp.float32)

    cls = cls_ref[...]
    one = jnp.float32(1.0)
    zero = jnp.float32(0.0)
    p = predt_ref[...]
    t = tgt_ref[pl.ds(i, 1), :].astype(jnp.float32)
    m = jnp.max(p, axis=0, keepdims=True)
    lab = jnp.min(jnp.where(p == m, cls, float(C)), axis=0, keepdims=True)
    oh_l = jnp.where(cls == lab, one, zero)
    oh_t = jnp.where(cls == t, one, zero)
    acc_ref[...] += jax.lax.dot_general(
        oh_t, oh_l, (((1,), (1,)), ((), ())),
        preferred_element_type=jnp.float32)

    @pl.when(i == nb - 1)
    def _finish():
        h = acc_ref[...]
        hist_ref[...] = h.astype(jnp.int32)
        r = jax.lax.broadcasted_iota(jnp.int32, (C, C), 0).astype(jnp.float32)
        c = jax.lax.broadcasted_iota(jnp.int32, (C, C), 1).astype(jnp.float32)
        eye = (r == c)
        dcol = jnp.sum(jnp.where(eye, h, 0.0), axis=1, keepdims=True)
        drow = jnp.sum(jnp.where(eye, h, 0.0), axis=0, keepdims=True)
        wins = (drow < dcol) | ((drow == dcol) & (c < r))
        rank = jnp.sum(jnp.where(wins, 1.0, 0.0), axis=1, keepdims=True)
        q = jnp.where(rank == c, 1.0, 0.0)
        vals_ref[...] = jnp.sum(q * dcol, axis=0, keepdims=True).astype(jnp.int32)
        keys_ref[...] = jnp.sum(q * r, axis=0, keepdims=True).astype(jnp.int32)


def kernel(pred, target):
    n, n_class = pred.shape
    b = 32768
    nb = (n + b - 1) // b
    npad = nb * b
    tgt2 = jnp.pad(target.astype(jnp.int32), (0, npad - n),
                   constant_values=-1).reshape(nb, b)
    predt = pred.T
    hist, keys, vals = pl.pallas_call(
        _conf_kernel,
        grid=(nb,),
        in_specs=[
            pl.BlockSpec((n_class, b), lambda i: (0, i)),
            pl.BlockSpec((nb, b), lambda i: (0, 0)),
        ],
        out_specs=[
            pl.BlockSpec((n_class, n_class), lambda i: (0, 0)),
            pl.BlockSpec((1, n_class), lambda i: (0, 0)),
            pl.BlockSpec((1, n_class), lambda i: (0, 0)),
        ],
        out_shape=[
            jax.ShapeDtypeStruct((n_class, n_class), jnp.int32),
            jax.ShapeDtypeStruct((1, n_class), jnp.int32),
            jax.ShapeDtypeStruct((1, n_class), jnp.int32),
        ],
        scratch_shapes=[
            pltpu.VMEM((n_class, n_class), jnp.float32),
            pltpu.VMEM((n_class, b), jnp.float32),
        ],
        compiler_params=pltpu.CompilerParams(
            dimension_semantics=("arbitrary",),
            fuse_transposed_lhs_in_matmul=True),
    )(predt, tgt2)
    return hist, keys.reshape(n_class), vals.reshape(n_class)


# speedup vs baseline: 6.5316x; 1.0411x over previous
import jax
import jax.numpy as jnp
from jax.experimental import pallas as pl
from jax.experimental.pallas import tpu as pltpu


def _conf_kernel(predt_ref, tgt_ref, hist_ref, keys_ref, vals_ref,
                 acc_ref, cls_ref):
    i = pl.program_id(0)
    nb = pl.num_programs(0)
    C, B = predt_ref.shape

    @pl.when(i == 0)
    def _init():
        acc_ref[...] = jnp.zeros_like(acc_ref)
        cls_ref[...] = jax.lax.broadcasted_iota(
            jnp.int32, (C, B), 0).astype(jnp.float32)

    cls = cls_ref[...]
    one = jnp.float32(1.0)
    zero = jnp.float32(0.0)
    p = predt_ref[...]
    t = tgt_ref[pl.ds(i, 1), :].astype(jnp.float32)
    m = jnp.max(p, axis=0, keepdims=True)
    lab = jnp.min(jnp.where(p == m, cls, float(C)), axis=0, keepdims=True)
    oh_l = jnp.where(cls == lab, one, zero)
    oh_t = jnp.where(cls == t, one, zero)
    acc_ref[...] += jax.lax.dot_general(
        oh_t, oh_l, (((1,), (1,)), ((), ())),
        preferred_element_type=jnp.float32)

    @pl.when(i == nb - 1)
    def _finish():
        h = acc_ref[...]
        hist_ref[...] = h.astype(jnp.int32)
        r = jax.lax.broadcasted_iota(jnp.int32, (C, C), 0).astype(jnp.float32)
        c = jax.lax.broadcasted_iota(jnp.int32, (C, C), 1).astype(jnp.float32)
        eye = (r == c)
        dcol = jnp.sum(jnp.where(eye, h, 0.0), axis=1, keepdims=True)
        drow = jnp.sum(jnp.where(eye, h, 0.0), axis=0, keepdims=True)
        wins = (drow < dcol) | ((drow == dcol) & (c < r))
        rank = jnp.sum(jnp.where(wins, 1.0, 0.0), axis=1, keepdims=True)
        q = jnp.where(rank == c, 1.0, 0.0)
        vals_ref[...] = jnp.sum(q * dcol, axis=0, keepdims=True).astype(jnp.int32)
        keys_ref[...] = jnp.sum(q * r, axis=0, keepdims=True).astype(jnp.int32)


def kernel(pred, target):
    n, n_class = pred.shape
    b = 65536
    nb = (n + b - 1) // b
    npad = nb * b
    tgt2 = jnp.pad(target.astype(jnp.int32), (0, npad - n),
                   constant_values=-1).reshape(nb, b)
    predt = pred.T
    hist, keys, vals = pl.pallas_call(
        _conf_kernel,
        grid=(nb,),
        in_specs=[
            pl.BlockSpec((n_class, b), lambda i: (0, i)),
            pl.BlockSpec((nb, b), lambda i: (0, 0)),
        ],
        out_specs=[
            pl.BlockSpec((n_class, n_class), lambda i: (0, 0)),
            pl.BlockSpec((1, n_class), lambda i: (0, 0)),
            pl.BlockSpec((1, n_class), lambda i: (0, 0)),
        ],
        out_shape=[
            jax.ShapeDtypeStruct((n_class, n_class), jnp.int32),
            jax.ShapeDtypeStruct((1, n_class), jnp.int32),
            jax.ShapeDtypeStruct((1, n_class), jnp.int32),
        ],
        scratch_shapes=[
            pltpu.VMEM((n_class, n_class), jnp.float32),
            pltpu.VMEM((n_class, b), jnp.float32),
        ],
        compiler_params=pltpu.CompilerParams(
            dimension_semantics=("arbitrary",),
            fuse_transposed_lhs_in_matmul=True),
    )(predt, tgt2)
    return hist, keys.reshape(n_class), vals.reshape(n_class)
